# baseline (speedup 1.0000x reference)
import functools

import jax
import jax.numpy as jnp
import numpy as np
from jax import lax
from jax.experimental import pallas as pl
from jax.experimental.pallas import tpu as pltpu
from jax.experimental.pallas import tpu_sc as plsc

SEQ_LEN = 1024
N_BINS = 64
RB = 32
K_CUT = 48
BIG = np.float32(1e30)
OFF = 8


def _edge_arrays():
    e = np.linspace(2.0, 22.0, N_BINS).astype(np.float32)[1:]
    elo = np.empty((N_BINS,), dtype=np.float32)
    elo[0] = -BIG
    elo[1:] = e
    ehi = np.empty((N_BINS,), dtype=np.float32)
    ehi[:63] = e
    ehi[63] = BIG
    return elo, ehi


def _sc_digitize_body(d_hbm, elo_hbm, ehi_hbm, out_hbm, d_v, elo_v, ehi_v,
                      res_v, bcast_v, sem):
    cid = lax.axis_index("c")
    sid = lax.axis_index("s")
    wid = sid * 2 + cid

    pltpu.sync_copy(d_hbm, d_v.at[pl.ds(OFF, SEQ_LEN)])
    pltpu.sync_copy(elo_hbm, elo_v)
    pltpu.sync_copy(ehi_hbm, ehi_v)

    elo_c = [elo_v[pl.ds(c * 16, 16)] for c in range(4)]
    ehi_c = [ehi_v[pl.ds(c * 16, 16)] for c in range(4)]

    lo_acc = [jnp.zeros((16,), jnp.float32) for _ in range(4)]
    hi_acc = [jnp.zeros((16,), jnp.float32) for _ in range(4)]
    for kk in range(K_CUT):
        dkv = plsc.load_gather(d_v, [jnp.full((16,), kk + OFF, jnp.int32)])
        for c in range(4):
            lo_acc[c] = lo_acc[c] + (dkv < elo_c[c]).astype(jnp.float32)
            hi_acc[c] = hi_acc[c] + (dkv < ehi_c[c]).astype(jnp.float32)
    rest = jnp.float32(SEQ_LEN - K_CUT)
    for c in range(4):
        hi_acc[c] = hi_acc[c] + jnp.where(
            ehi_c[c] == BIG, rest, jnp.float32(0.0)
        )
        res_v[pl.ds(OFF + c * 16, 16)] = lo_acc[c]
        res_v[pl.ds(OFF + 64 + c * 16, 16)] = hi_acc[c]

    @pl.when(wid == 0)
    def _():
        for b in range(N_BINS):
            lo_b = plsc.load_gather(
                res_v, [jnp.full((16,), OFF + b, jnp.int32)]
            )
            hi_b = plsc.load_gather(
                res_v, [jnp.full((16,), OFF + 64 + b, jnp.int32)]
            )
            for c in range(8):
                bcast_v[0, b, pl.ds(c * 16, 16)] = lo_b
                bcast_v[1, b, pl.ds(c * 16, 16)] = hi_b
        pltpu.sync_copy(bcast_v, out_hbm)


def _tc_onehot_body(cnt_ref, out_ref):
    lo4 = cnt_ref[0, :, 0:1].reshape(1, 1, N_BINS, 1)
    hi4 = cnt_ref[1, :, 0:1].reshape(1, 1, N_BINS, 1)
    r = pl.program_id(0)
    row = jax.lax.broadcasted_iota(jnp.int32, (1, RB, 1, SEQ_LEN), 1)
    col = jax.lax.broadcasted_iota(jnp.int32, (1, RB, 1, SEQ_LEN), 3)
    sep = jnp.abs(row + (r * RB) - col).astype(jnp.float32)
    cond = (sep >= lo4) & (sep < hi4)
    out_ref[...] = jnp.where(cond, jnp.float32(10.0), jnp.float32(-10.0))


@jax.jit
def _logits(d, elo, ehi):
    mesh = plsc.VectorSubcoreMesh(core_axis_name="c", subcore_axis_name="s")
    digitize = functools.partial(
        pl.kernel,
        mesh=mesh,
        out_type=jax.ShapeDtypeStruct((2, N_BINS, 128), jnp.float32),
        scratch_types=[
            pltpu.VMEM((SEQ_LEN + OFF,), jnp.float32),
            pltpu.VMEM((N_BINS,), jnp.float32),
            pltpu.VMEM((N_BINS,), jnp.float32),
            pltpu.VMEM((OFF + 2 * N_BINS,), jnp.float32),
            pltpu.VMEM((2, N_BINS, 128), jnp.float32),
            pltpu.SemaphoreType.DMA,
        ],
        compiler_params=pltpu.CompilerParams(
            needs_layout_passes=False, use_tc_tiling_on_sc=False
        ),
    )(_sc_digitize_body)
    cnt = digitize(d, elo, ehi)

    out = pl.pallas_call(
        _tc_onehot_body,
        grid=(SEQ_LEN // RB,),
        in_specs=[pl.BlockSpec((2, N_BINS, 128), lambda i: (0, 0, 0))],
        out_specs=pl.BlockSpec(
            (1, RB, N_BINS, SEQ_LEN), lambda i: (0, i, 0, 0)
        ),
        out_shape=jax.ShapeDtypeStruct(
            (1, SEQ_LEN, N_BINS, SEQ_LEN), jnp.float32
        ),
    )(cnt)
    return jnp.swapaxes(out, 2, 3)


def kernel(x):
    B, L, _ = x.shape
    k = jnp.arange(SEQ_LEN + 2, dtype=jnp.float32)
    sep_to_dist = jnp.clip(2.0 + 2.5 * jnp.power(k, 0.55), 2.0, 22.0)
    elo, ehi = _edge_arrays()
    return _logits(sep_to_dist[:SEQ_LEN], jnp.asarray(elo), jnp.asarray(ehi))
